# Initial kernel scaffold; baseline (speedup 1.0000x reference)
import jax
import jax.numpy as jnp
from jax.experimental import pallas as pl


def kernel(loc_data, conf_data, targets, priors):
    raise NotImplementedError("write your pallas kernel here")


# speedup vs baseline: 13.0972x; 13.0972x over previous
import functools

import jax
import jax.numpy as jnp
from jax.experimental import pallas as pl
from jax.experimental.pallas import tpu as pltpu

_OVERLAP_TH = 0.35
_VAR0, _VAR1 = 0.1, 0.2
_NP_RATIO = 3
_LANES = 128


def _body(loc_ref, conf_ref, tgt_ref, pln_ref, out_ref, *, n_obj, p_real, rows):
    i = pl.program_id(0)

    px1 = pln_ref[0]
    py1 = pln_ref[1]
    px2 = pln_ref[2]
    py2 = pln_ref[3]
    pcx = pln_ref[4]
    pcy = pln_ref[5]
    pw = pln_ref[6]
    ph = pln_ref[7]
    parea = pln_ref[8]

    lane = jax.lax.broadcasted_iota(jnp.int32, (rows, _LANES), 1)
    row = jax.lax.broadcasted_iota(jnp.int32, (rows, _LANES), 0)
    lin = row * _LANES + lane
    valid = lin < p_real

    best_ov = jnp.full((rows, _LANES), -2.0, jnp.float32)
    m1 = jnp.zeros((rows, _LANES), jnp.float32)
    m2 = jnp.zeros((rows, _LANES), jnp.float32)
    m3 = jnp.ones((rows, _LANES), jnp.float32)
    m4 = jnp.ones((rows, _LANES), jnp.float32)
    lab = jnp.zeros((rows, _LANES), jnp.float32)
    best_prior = []

    for j in range(n_obj):
        tx1 = tgt_ref[0, j, 0]
        ty1 = tgt_ref[0, j, 1]
        tx2 = tgt_ref[0, j, 2]
        ty2 = tgt_ref[0, j, 3]
        tl = tgt_ref[0, j, 4]
        iw = jnp.maximum(jnp.minimum(tx2, px2) - jnp.maximum(tx1, px1), 0.0)
        ih = jnp.maximum(jnp.minimum(ty2, py2) - jnp.maximum(ty1, py1), 0.0)
        inter = iw * ih
        tarea = (tx2 - tx1) * (ty2 - ty1)
        iou = inter / (tarea + parea - inter)
        iou = jnp.where(valid, iou, -1.0)
        upd = iou > best_ov
        best_ov = jnp.where(upd, iou, best_ov)
        m1 = jnp.where(upd, tx1, m1)
        m2 = jnp.where(upd, ty1, m2)
        m3 = jnp.where(upd, tx2, m3)
        m4 = jnp.where(upd, ty2, m4)
        lab = jnp.where(upd, tl, lab)
        mj = jnp.max(iou)
        best_prior.append(jnp.min(jnp.where(iou == mj, lin, p_real)))

    for j in range(n_obj):
        f = lin == best_prior[j]
        tx1 = tgt_ref[0, j, 0]
        ty1 = tgt_ref[0, j, 1]
        tx2 = tgt_ref[0, j, 2]
        ty2 = tgt_ref[0, j, 3]
        tl = tgt_ref[0, j, 4]
        best_ov = jnp.where(f, 2.0, best_ov)
        m1 = jnp.where(f, tx1, m1)
        m2 = jnp.where(f, ty1, m2)
        m3 = jnp.where(f, tx2, m3)
        m4 = jnp.where(f, ty2, m4)
        lab = jnp.where(f, tl, lab)

    conf = jnp.where(best_ov < _OVERLAP_TH, 0.0, lab)
    pos = conf > 0.0
    npos = jnp.sum(pos.astype(jnp.int32))

    gcx = ((m1 + m3) * 0.5 - pcx) / (_VAR0 * pw)
    gcy = ((m2 + m4) * 0.5 - pcy) / (_VAR0 * ph)
    gw = jnp.log((m3 - m1) / pw) / _VAR1
    gh = jnp.log((m4 - m2) / ph) / _VAR1

    def sl1(d):
        ad = jnp.abs(d)
        return jnp.where(ad < 1.0, 0.5 * d * d, ad - 0.5)

    hub = (sl1(loc_ref[0, 0] - gcx) + sl1(loc_ref[0, 1] - gcy)
           + sl1(loc_ref[0, 2] - gw) + sl1(loc_ref[0, 3] - gh))
    loss_l = jnp.sum(jnp.where(pos, hub, 0.0))

    c0 = conf_ref[0, 0]
    c1 = conf_ref[0, 1]
    mx = jnp.maximum(c0, c1)
    lse = mx + jnp.log(1.0 + jnp.exp(jnp.minimum(c0, c1) - mx))
    ce_pos = jnp.sum(jnp.where(pos, lse - c1, 0.0))

    rank = jnp.where(valid & jnp.logical_not(pos), lse - c0, 0.0)
    rb = jax.lax.bitcast_convert_type(rank, jnp.int32)

    kk = jnp.minimum(_NP_RATIO * npos, p_real - 1)

    def bit_step(b, prefix):
        cand = prefix | (jnp.int32(1) << (30 - b))
        cnt = jnp.sum((rb >= cand).astype(jnp.int32))
        return jnp.where(cnt >= kk, cand, prefix)

    prefix = jax.lax.fori_loop(0, 31, bit_step, jnp.int32(0))
    t = jax.lax.bitcast_convert_type(prefix, jnp.float32)
    gt = rb > prefix
    cnt_gt = jnp.sum(gt.astype(jnp.float32))
    sum_gt = jnp.sum(jnp.where(gt, rank, 0.0))
    neg_sum = jnp.where(kk > 0,
                        sum_gt + (kk.astype(jnp.float32) - cnt_gt) * t,
                        0.0)

    @pl.when(i == 0)
    def _():
        out_ref[0] = 0.0
        out_ref[1] = 0.0
        out_ref[2] = 0.0
        out_ref[3] = 0.0

    out_ref[0] += loss_l
    out_ref[1] += ce_pos
    out_ref[2] += neg_sum
    out_ref[3] += npos.astype(jnp.float32)


def kernel(loc_data, conf_data, targets, priors):
    B, P, _ = loc_data.shape
    n_obj = targets.shape[1]
    rows = (P + _LANES - 1) // _LANES
    rows = ((rows + 7) // 8) * 8
    pp = rows * _LANES
    pad = pp - P

    pad_cf = jnp.tile(jnp.asarray([[0.5, 0.5, 1.0, 1.0]], priors.dtype), (pad, 1))
    pr = jnp.concatenate([priors, pad_cf], axis=0)
    pf = jnp.concatenate([pr[:, :2] - pr[:, 2:] / 2.0,
                          pr[:, :2] + pr[:, 2:] / 2.0], axis=1)
    parea = (pf[:, 2] - pf[:, 0]) * (pf[:, 3] - pf[:, 1])
    planes = jnp.concatenate([pf.T, pr.T, parea[None, :]], axis=0)
    planes = planes.reshape(9, rows, _LANES)

    loc_p = jnp.pad(jnp.moveaxis(loc_data, -1, 1), ((0, 0), (0, 0), (0, pad)))
    loc_p = loc_p.reshape(B, 4, rows, _LANES)
    conf_p = jnp.pad(jnp.moveaxis(conf_data, -1, 1), ((0, 0), (0, 0), (0, pad)))
    conf_p = conf_p.reshape(B, 2, rows, _LANES)

    body = functools.partial(_body, n_obj=n_obj, p_real=P, rows=rows)
    acc = pl.pallas_call(
        body,
        grid=(B,),
        in_specs=[
            pl.BlockSpec((1, 4, rows, _LANES), lambda i: (i, 0, 0, 0)),
            pl.BlockSpec((1, 2, rows, _LANES), lambda i: (i, 0, 0, 0)),
            pl.BlockSpec((1, n_obj, 5), lambda i: (i, 0, 0),
                         memory_space=pltpu.SMEM),
            pl.BlockSpec((9, rows, _LANES), lambda i: (0, 0, 0)),
        ],
        out_specs=pl.BlockSpec((4,), lambda i: (0,), memory_space=pltpu.SMEM),
        out_shape=jax.ShapeDtypeStruct((4,), jnp.float32),
    )(loc_p, conf_p, targets, planes)

    n = jnp.maximum(acc[3], 1.0)
    return acc[0] / n, (acc[1] + acc[2]) / n


# speedup vs baseline: 19.9327x; 1.5219x over previous
import functools

import jax
import jax.numpy as jnp
from jax.experimental import pallas as pl
from jax.experimental.pallas import tpu as pltpu

_OVERLAP_TH = 0.35
_VAR0, _VAR1 = 0.1, 0.2
_NP_RATIO = 3
_LANES = 128


def _body(loc_ref, conf_ref, tgt_ref, pln_ref, out_ref, rank_ref, kk_ref,
          pref_ref, *, n_obj, p_real, rows, num_images):
    i = pl.program_id(0)

    px1 = pln_ref[0]
    py1 = pln_ref[1]
    px2 = pln_ref[2]
    py2 = pln_ref[3]
    parea = pln_ref[8]

    best_ov = jnp.full((rows, _LANES), -2.0, jnp.float32)
    best_j = jnp.zeros((rows, _LANES), jnp.int32)
    best_prior = []

    for j in range(n_obj):
        tx1 = tgt_ref[0, j, 0]
        ty1 = tgt_ref[0, j, 1]
        tx2 = tgt_ref[0, j, 2]
        ty2 = tgt_ref[0, j, 3]
        iw = jnp.maximum(jnp.minimum(tx2, px2) - jnp.maximum(tx1, px1), 0.0)
        ih = jnp.maximum(jnp.minimum(ty2, py2) - jnp.maximum(ty1, py1), 0.0)
        inter = iw * ih
        tarea = (tx2 - tx1) * (ty2 - ty1)
        iou = inter / (tarea + parea - inter)
        upd = iou > best_ov
        best_ov = jnp.where(upd, iou, best_ov)
        best_j = jnp.where(upd, j, best_j)
        best_prior.append(iou)

    lane = jax.lax.broadcasted_iota(jnp.int32, (rows, _LANES), 1)
    row = jax.lax.broadcasted_iota(jnp.int32, (rows, _LANES), 0)
    lin = row * _LANES + lane

    bp_idx = []
    for j in range(n_obj):
        iou = best_prior[j]
        mj = jnp.max(iou)
        bp_idx.append(jnp.min(jnp.where(iou == mj, lin, p_real)))

    for j in range(n_obj):
        f = lin == bp_idx[j]
        best_ov = jnp.where(f, 2.0, best_ov)
        best_j = jnp.where(f, j, best_j)

    m1 = jnp.zeros((rows, _LANES), jnp.float32)
    m2 = jnp.zeros((rows, _LANES), jnp.float32)
    m3 = jnp.ones((rows, _LANES), jnp.float32)
    m4 = jnp.ones((rows, _LANES), jnp.float32)
    lab = jnp.zeros((rows, _LANES), jnp.float32)
    for j in range(n_obj):
        s = best_j == j
        m1 = jnp.where(s, tgt_ref[0, j, 0], m1)
        m2 = jnp.where(s, tgt_ref[0, j, 1], m2)
        m3 = jnp.where(s, tgt_ref[0, j, 2], m3)
        m4 = jnp.where(s, tgt_ref[0, j, 3], m4)
        lab = jnp.where(s, tgt_ref[0, j, 4], lab)

    conf = jnp.where(best_ov < _OVERLAP_TH, 0.0, lab)
    pos = conf > 0.0
    npos = jnp.sum(pos.astype(jnp.int32))

    pcx = pln_ref[4]
    pcy = pln_ref[5]
    pw = pln_ref[6]
    ph = pln_ref[7]
    gcx = ((m1 + m3) * 0.5 - pcx) / (_VAR0 * pw)
    gcy = ((m2 + m4) * 0.5 - pcy) / (_VAR0 * ph)
    gw = jnp.log((m3 - m1) / pw) / _VAR1
    gh = jnp.log((m4 - m2) / ph) / _VAR1

    def sl1(d):
        ad = jnp.abs(d)
        return jnp.where(ad < 1.0, 0.5 * d * d, ad - 0.5)

    hub = (sl1(loc_ref[0, 0] - gcx) + sl1(loc_ref[0, 1] - gcy)
           + sl1(loc_ref[0, 2] - gw) + sl1(loc_ref[0, 3] - gh))
    loss_l = jnp.sum(jnp.where(pos, hub, 0.0))

    c0 = conf_ref[0, 0]
    c1 = conf_ref[0, 1]
    mx = jnp.maximum(c0, c1)
    lse = mx + jnp.log(1.0 + jnp.exp(jnp.minimum(c0, c1) - mx))
    ce_pos = jnp.sum(jnp.where(pos, lse - c1, 0.0))

    rank = jnp.where(pos, 0.0, lse - c0)
    rank_ref[i] = jax.lax.bitcast_convert_type(rank, jnp.int32)
    kk_ref[i] = jnp.minimum(_NP_RATIO * npos, p_real - 1)

    @pl.when(i == 0)
    def _():
        out_ref[0] = 0.0
        out_ref[1] = 0.0
        out_ref[2] = 0.0
        out_ref[3] = 0.0

    out_ref[0] += loss_l
    out_ref[1] += ce_pos
    out_ref[3] += npos.astype(jnp.float32)

    @pl.when(i == num_images - 1)
    def _():
        for im in range(num_images):
            pref_ref[im] = 0

        def bit_step(b, carry):
            bit = jnp.int32(1) << (30 - b)
            for im in range(num_images):
                pref = pref_ref[im]
                cand = pref | bit
                cnt = jnp.sum((rank_ref[im] >= cand).astype(jnp.int32))
                pref_ref[im] = jnp.where(cnt >= kk_ref[im], cand, pref)
            return carry

        jax.lax.fori_loop(0, 31, bit_step, jnp.int32(0))

        neg_total = jnp.float32(0.0)
        for im in range(num_images):
            pref = pref_ref[im]
            rbv = rank_ref[im]
            gt = rbv > pref
            cnt_gt = jnp.sum(gt.astype(jnp.float32))
            rv = jax.lax.bitcast_convert_type(rbv, jnp.float32)
            sum_gt = jnp.sum(jnp.where(gt, rv, 0.0))
            t = jax.lax.bitcast_convert_type(pref, jnp.float32)
            kkf = kk_ref[im].astype(jnp.float32)
            neg_total += jnp.where(kk_ref[im] > 0,
                                   sum_gt + (kkf - cnt_gt) * t, 0.0)
        out_ref[2] = neg_total


def kernel(loc_data, conf_data, targets, priors):
    B, P, _ = loc_data.shape
    n_obj = targets.shape[1]
    rows = (P + _LANES - 1) // _LANES
    rows = ((rows + 7) // 8) * 8
    pp = rows * _LANES
    pad = pp - P

    pad_cf = jnp.tile(jnp.asarray([[2.5, 2.5, 1.0, 1.0]], priors.dtype), (pad, 1))
    pr = jnp.concatenate([priors, pad_cf], axis=0)
    pf = jnp.concatenate([pr[:, :2] - pr[:, 2:] / 2.0,
                          pr[:, :2] + pr[:, 2:] / 2.0], axis=1)
    parea = (pf[:, 2] - pf[:, 0]) * (pf[:, 3] - pf[:, 1])
    planes = jnp.concatenate([pf.T, pr.T, parea[None, :]], axis=0)
    planes = planes.reshape(9, rows, _LANES)

    loc_p = jnp.pad(jnp.moveaxis(loc_data, -1, 1), ((0, 0), (0, 0), (0, pad)))
    loc_p = loc_p.reshape(B, 4, rows, _LANES)
    conf_pad = jnp.broadcast_to(
        jnp.asarray([40.0, -40.0], conf_data.dtype)[None, :, None], (B, 2, pad))
    conf_p = jnp.concatenate([jnp.moveaxis(conf_data, -1, 1), conf_pad], axis=2)
    conf_p = conf_p.reshape(B, 2, rows, _LANES)

    body = functools.partial(_body, n_obj=n_obj, p_real=P, rows=rows,
                             num_images=B)
    acc = pl.pallas_call(
        body,
        grid=(B,),
        in_specs=[
            pl.BlockSpec((1, 4, rows, _LANES), lambda i: (i, 0, 0, 0)),
            pl.BlockSpec((1, 2, rows, _LANES), lambda i: (i, 0, 0, 0)),
            pl.BlockSpec((1, n_obj, 5), lambda i: (i, 0, 0),
                         memory_space=pltpu.SMEM),
            pl.BlockSpec((9, rows, _LANES), lambda i: (0, 0, 0)),
        ],
        out_specs=pl.BlockSpec((4,), lambda i: (0,), memory_space=pltpu.SMEM),
        out_shape=jax.ShapeDtypeStruct((4,), jnp.float32),
        scratch_shapes=[
            pltpu.VMEM((B, rows, _LANES), jnp.int32),
            pltpu.SMEM((B,), jnp.int32),
            pltpu.SMEM((B,), jnp.int32),
        ],
    )(loc_p, conf_p, targets, planes)

    n = jnp.maximum(acc[3], 1.0)
    return acc[0] / n, (acc[1] + acc[2]) / n


# speedup vs baseline: 20.8264x; 1.0448x over previous
import functools

import jax
import jax.numpy as jnp
from jax.experimental import pallas as pl
from jax.experimental.pallas import tpu as pltpu

_OVERLAP_TH = 0.35
_VAR0, _VAR1 = 0.1, 0.2
_NP_RATIO = 3
_LANES = 128


def _body(loc_ref, conf_ref, tgt_ref, pln_ref, out_ref, rank_ref, kk_ref,
          pref_ref, *, n_obj, p_real, rows, num_images):
    i = pl.program_id(0)

    px1 = pln_ref[0]
    py1 = pln_ref[1]
    px2 = pln_ref[2]
    py2 = pln_ref[3]
    parea = pln_ref[8]

    lane = jax.lax.broadcasted_iota(jnp.int32, (rows, _LANES), 1)
    row = jax.lax.broadcasted_iota(jnp.int32, (rows, _LANES), 0)
    lin = row * _LANES + lane

    best_ov = jnp.full((rows, _LANES), -2.0, jnp.float32)
    best_j = jnp.zeros((rows, _LANES), jnp.int32)
    bp_idx = []

    for j in range(n_obj):
        tx1 = tgt_ref[0, j, 0]
        ty1 = tgt_ref[0, j, 1]
        tx2 = tgt_ref[0, j, 2]
        ty2 = tgt_ref[0, j, 3]
        iw = jnp.maximum(jnp.minimum(tx2, px2) - jnp.maximum(tx1, px1), 0.0)
        ih = jnp.maximum(jnp.minimum(ty2, py2) - jnp.maximum(ty1, py1), 0.0)
        inter = iw * ih
        tarea = (tx2 - tx1) * (ty2 - ty1)
        iou = inter / (tarea + parea - inter)
        upd = iou > best_ov
        best_ov = jnp.where(upd, iou, best_ov)
        best_j = jnp.where(upd, j, best_j)
        mj = jnp.max(iou)
        bp_idx.append(jnp.min(jnp.where(iou == mj, lin, p_real)))

    for j in range(n_obj):
        f = lin == bp_idx[j]
        best_ov = jnp.where(f, 2.0, best_ov)
        best_j = jnp.where(f, j, best_j)

    m1 = jnp.zeros((rows, _LANES), jnp.float32)
    m2 = jnp.zeros((rows, _LANES), jnp.float32)
    m3 = jnp.ones((rows, _LANES), jnp.float32)
    m4 = jnp.ones((rows, _LANES), jnp.float32)
    lab = jnp.zeros((rows, _LANES), jnp.float32)
    for j in range(n_obj):
        s = best_j == j
        m1 = jnp.where(s, tgt_ref[0, j, 0], m1)
        m2 = jnp.where(s, tgt_ref[0, j, 1], m2)
        m3 = jnp.where(s, tgt_ref[0, j, 2], m3)
        m4 = jnp.where(s, tgt_ref[0, j, 3], m4)
        lab = jnp.where(s, tgt_ref[0, j, 4], lab)

    conf = jnp.where(best_ov < _OVERLAP_TH, 0.0, lab)
    pos = conf > 0.0
    npos = jnp.sum(pos.astype(jnp.int32))

    pcx = pln_ref[4]
    pcy = pln_ref[5]
    pw = pln_ref[6]
    ph = pln_ref[7]
    gcx = ((m1 + m3) * 0.5 - pcx) / (_VAR0 * pw)
    gcy = ((m2 + m4) * 0.5 - pcy) / (_VAR0 * ph)
    gw = jnp.log((m3 - m1) / pw) / _VAR1
    gh = jnp.log((m4 - m2) / ph) / _VAR1

    def sl1(d):
        ad = jnp.abs(d)
        return jnp.where(ad < 1.0, 0.5 * d * d, ad - 0.5)

    hub = (sl1(loc_ref[0, 0] - gcx) + sl1(loc_ref[0, 1] - gcy)
           + sl1(loc_ref[0, 2] - gw) + sl1(loc_ref[0, 3] - gh))
    loss_l = jnp.sum(jnp.where(pos, hub, 0.0))

    c0 = conf_ref[0, 0]
    c1 = conf_ref[0, 1]
    mx = jnp.maximum(c0, c1)
    lse = mx + jnp.log(1.0 + jnp.exp(jnp.minimum(c0, c1) - mx))
    ce_pos = jnp.sum(jnp.where(pos, lse - c1, 0.0))

    rank = jnp.where(pos, 0.0, lse - c0)
    rank_ref[i] = jax.lax.bitcast_convert_type(rank, jnp.int32)
    kk_ref[i] = jnp.minimum(_NP_RATIO * npos, p_real - 1)

    @pl.when(i == 0)
    def _():
        out_ref[0] = 0.0
        out_ref[1] = 0.0
        out_ref[2] = 0.0
        out_ref[3] = 0.0

    out_ref[0] += loss_l
    out_ref[1] += ce_pos
    out_ref[3] += npos.astype(jnp.float32)

    @pl.when(i == num_images - 1)
    def _():
        for im in range(num_images):
            pref_ref[im] = 0

        def bit_step(b, carry):
            bit = jnp.int32(1) << (30 - b)
            for im in range(num_images):
                pref = pref_ref[im]
                cand = pref | bit
                cnt = jnp.sum((rank_ref[im] >= cand).astype(jnp.int32))
                pref_ref[im] = jnp.where(cnt >= kk_ref[im], cand, pref)
            return carry

        jax.lax.fori_loop(0, 31, bit_step, jnp.int32(0))

        neg_total = jnp.float32(0.0)
        for im in range(num_images):
            pref = pref_ref[im]
            rbv = rank_ref[im]
            gt = rbv > pref
            cnt_gt = jnp.sum(gt.astype(jnp.float32))
            rv = jax.lax.bitcast_convert_type(rbv, jnp.float32)
            sum_gt = jnp.sum(jnp.where(gt, rv, 0.0))
            t = jax.lax.bitcast_convert_type(pref, jnp.float32)
            kkf = kk_ref[im].astype(jnp.float32)
            neg_total += jnp.where(kk_ref[im] > 0,
                                   sum_gt + (kkf - cnt_gt) * t, 0.0)
        out_ref[2] = neg_total


def kernel(loc_data, conf_data, targets, priors):
    B, P, _ = loc_data.shape
    n_obj = targets.shape[1]
    rows = (P + _LANES - 1) // _LANES
    rows = ((rows + 7) // 8) * 8
    pp = rows * _LANES
    pad = pp - P

    pad_cf = jnp.tile(jnp.asarray([[2.5, 2.5, 1.0, 1.0]], priors.dtype), (pad, 1))
    pr = jnp.concatenate([priors, pad_cf], axis=0)
    pf = jnp.concatenate([pr[:, :2] - pr[:, 2:] / 2.0,
                          pr[:, :2] + pr[:, 2:] / 2.0], axis=1)
    parea = (pf[:, 2] - pf[:, 0]) * (pf[:, 3] - pf[:, 1])
    planes = jnp.concatenate([pf.T, pr.T, parea[None, :]], axis=0)
    planes = planes.reshape(9, rows, _LANES)

    loc_p = jnp.pad(jnp.moveaxis(loc_data, -1, 1), ((0, 0), (0, 0), (0, pad)))
    loc_p = loc_p.reshape(B, 4, rows, _LANES)
    conf_pad = jnp.broadcast_to(
        jnp.asarray([40.0, -40.0], conf_data.dtype)[None, :, None], (B, 2, pad))
    conf_p = jnp.concatenate([jnp.moveaxis(conf_data, -1, 1), conf_pad], axis=2)
    conf_p = conf_p.reshape(B, 2, rows, _LANES)

    body = functools.partial(_body, n_obj=n_obj, p_real=P, rows=rows,
                             num_images=B)
    acc = pl.pallas_call(
        body,
        grid=(B,),
        in_specs=[
            pl.BlockSpec((1, 4, rows, _LANES), lambda i: (i, 0, 0, 0)),
            pl.BlockSpec((1, 2, rows, _LANES), lambda i: (i, 0, 0, 0)),
            pl.BlockSpec((1, n_obj, 5), lambda i: (i, 0, 0),
                         memory_space=pltpu.SMEM),
            pl.BlockSpec((9, rows, _LANES), lambda i: (0, 0, 0)),
        ],
        out_specs=pl.BlockSpec((4,), lambda i: (0,), memory_space=pltpu.SMEM),
        out_shape=jax.ShapeDtypeStruct((4,), jnp.float32),
        scratch_shapes=[
            pltpu.VMEM((B, rows, _LANES), jnp.int32),
            pltpu.SMEM((B,), jnp.int32),
            pltpu.SMEM((B,), jnp.int32),
        ],
    )(loc_p, conf_p, targets, planes)

    n = jnp.maximum(acc[3], 1.0)
    return acc[0] / n, (acc[1] + acc[2]) / n


# speedup vs baseline: 21.2387x; 1.0198x over previous
import functools

import jax
import jax.numpy as jnp
from jax.experimental import pallas as pl
from jax.experimental.pallas import tpu as pltpu

_OVERLAP_TH = 0.35
_VAR0, _VAR1 = 0.1, 0.2
_NP_RATIO = 3
_LANES = 128
_IMGS_PER_STEP = 2


def _body(loc_ref, conf_ref, tgt_ref, pln_ref, out_ref, rank_ref, kk_ref,
          pref_ref, *, n_obj, p_real, rows, num_images):
    step = pl.program_id(0)

    ones_row = jnp.ones((1, rows), jnp.float32)

    def msum(x):
        return jnp.sum(jax.lax.dot_general(
            ones_row, x, (((1,), (0,)), ((), ())),
            preferred_element_type=jnp.float32))

    px1 = pln_ref[0]
    py1 = pln_ref[1]
    px2 = pln_ref[2]
    py2 = pln_ref[3]
    parea = pln_ref[8]
    pcx = pln_ref[4]
    pcy = pln_ref[5]
    pw = pln_ref[6]
    ph = pln_ref[7]

    lane = jax.lax.broadcasted_iota(jnp.int32, (rows, _LANES), 1)
    row = jax.lax.broadcasted_iota(jnp.int32, (rows, _LANES), 0)
    lin = row * _LANES + lane

    @pl.when(step == 0)
    def _():
        out_ref[0] = 0.0
        out_ref[1] = 0.0
        out_ref[2] = 0.0
        out_ref[3] = 0.0

    for im in range(_IMGS_PER_STEP):
        best_ov = jnp.full((rows, _LANES), -2.0, jnp.float32)
        best_j = jnp.zeros((rows, _LANES), jnp.int32)
        bp_idx = []

        for j in range(n_obj):
            tx1 = tgt_ref[im, j, 0]
            ty1 = tgt_ref[im, j, 1]
            tx2 = tgt_ref[im, j, 2]
            ty2 = tgt_ref[im, j, 3]
            iw = jnp.maximum(jnp.minimum(tx2, px2) - jnp.maximum(tx1, px1), 0.0)
            ih = jnp.maximum(jnp.minimum(ty2, py2) - jnp.maximum(ty1, py1), 0.0)
            inter = iw * ih
            tarea = (tx2 - tx1) * (ty2 - ty1)
            iou = inter / (tarea + parea - inter)
            upd = iou > best_ov
            best_ov = jnp.where(upd, iou, best_ov)
            best_j = jnp.where(upd, j, best_j)
            mj = jnp.max(iou)
            bp_idx.append(jnp.min(jnp.where(iou == mj, lin, p_real)))

        for j in range(n_obj):
            f = lin == bp_idx[j]
            best_ov = jnp.where(f, 2.0, best_ov)
            best_j = jnp.where(f, j, best_j)

        m1 = jnp.zeros((rows, _LANES), jnp.float32)
        m2 = jnp.zeros((rows, _LANES), jnp.float32)
        m3 = jnp.ones((rows, _LANES), jnp.float32)
        m4 = jnp.ones((rows, _LANES), jnp.float32)
        lab = jnp.zeros((rows, _LANES), jnp.float32)
        for j in range(n_obj):
            s = best_j == j
            m1 = jnp.where(s, tgt_ref[im, j, 0], m1)
            m2 = jnp.where(s, tgt_ref[im, j, 1], m2)
            m3 = jnp.where(s, tgt_ref[im, j, 2], m3)
            m4 = jnp.where(s, tgt_ref[im, j, 3], m4)
            lab = jnp.where(s, tgt_ref[im, j, 4], lab)

        conf = jnp.where(best_ov < _OVERLAP_TH, 0.0, lab)
        pos = conf > 0.0
        npos = msum(pos.astype(jnp.float32))

        gcx = ((m1 + m3) * 0.5 - pcx) / (_VAR0 * pw)
        gcy = ((m2 + m4) * 0.5 - pcy) / (_VAR0 * ph)
        gw = jnp.log((m3 - m1) / pw) / _VAR1
        gh = jnp.log((m4 - m2) / ph) / _VAR1

        def sl1(d):
            ad = jnp.abs(d)
            return jnp.where(ad < 1.0, 0.5 * d * d, ad - 0.5)

        hub = (sl1(loc_ref[im, 0] - gcx) + sl1(loc_ref[im, 1] - gcy)
               + sl1(loc_ref[im, 2] - gw) + sl1(loc_ref[im, 3] - gh))
        loss_l = msum(jnp.where(pos, hub, 0.0))

        c0 = conf_ref[im, 0]
        c1 = conf_ref[im, 1]
        mx = jnp.maximum(c0, c1)
        lse = mx + jnp.log(1.0 + jnp.exp(jnp.minimum(c0, c1) - mx))
        ce_pos = msum(jnp.where(pos, lse - c1, 0.0))

        rank = jnp.where(pos, 0.0, lse - c0)
        img = step * _IMGS_PER_STEP + im
        rank_ref[img] = jax.lax.bitcast_convert_type(rank, jnp.int32)
        npos_i = npos.astype(jnp.int32)
        kk_ref[img] = jnp.minimum(_NP_RATIO * npos_i, p_real - 1)

        out_ref[0] += loss_l
        out_ref[1] += ce_pos
        out_ref[3] += npos

    @pl.when(step == num_images // _IMGS_PER_STEP - 1)
    def _():
        for im in range(num_images):
            pref_ref[im] = 0

        def bit_step(b, carry):
            bit = jnp.int32(1) << (30 - b)
            for im in range(num_images):
                pref = pref_ref[im]
                cand = pref | bit
                ge = (rank_ref[im] >= cand).astype(jnp.float32)
                cnt = msum(ge)
                pref_ref[im] = jnp.where(cnt >= kk_ref[im].astype(jnp.float32),
                                         cand, pref)
            return carry

        jax.lax.fori_loop(0, 31, bit_step, jnp.int32(0))

        neg_total = jnp.float32(0.0)
        for im in range(num_images):
            pref = pref_ref[im]
            rbv = rank_ref[im]
            gt = rbv > pref
            cnt_gt = msum(gt.astype(jnp.float32))
            rv = jax.lax.bitcast_convert_type(rbv, jnp.float32)
            sum_gt = msum(jnp.where(gt, rv, 0.0))
            t = jax.lax.bitcast_convert_type(pref, jnp.float32)
            kkf = kk_ref[im].astype(jnp.float32)
            neg_total += jnp.where(kk_ref[im] > 0,
                                   sum_gt + (kkf - cnt_gt) * t, 0.0)
        out_ref[2] = neg_total


def kernel(loc_data, conf_data, targets, priors):
    B, P, _ = loc_data.shape
    n_obj = targets.shape[1]
    rows = (P + _LANES - 1) // _LANES
    rows = ((rows + 7) // 8) * 8
    pp = rows * _LANES
    pad = pp - P

    pad_cf = jnp.tile(jnp.asarray([[2.5, 2.5, 1.0, 1.0]], priors.dtype), (pad, 1))
    pr = jnp.concatenate([priors, pad_cf], axis=0)
    pf = jnp.concatenate([pr[:, :2] - pr[:, 2:] / 2.0,
                          pr[:, :2] + pr[:, 2:] / 2.0], axis=1)
    parea = (pf[:, 2] - pf[:, 0]) * (pf[:, 3] - pf[:, 1])
    planes = jnp.concatenate([pf.T, pr.T, parea[None, :]], axis=0)
    planes = planes.reshape(9, rows, _LANES)

    loc_p = jnp.pad(jnp.moveaxis(loc_data, -1, 1), ((0, 0), (0, 0), (0, pad)))
    loc_p = loc_p.reshape(B, 4, rows, _LANES)
    conf_pad = jnp.broadcast_to(
        jnp.asarray([40.0, -40.0], conf_data.dtype)[None, :, None], (B, 2, pad))
    conf_p = jnp.concatenate([jnp.moveaxis(conf_data, -1, 1), conf_pad], axis=2)
    conf_p = conf_p.reshape(B, 2, rows, _LANES)

    g = _IMGS_PER_STEP
    body = functools.partial(_body, n_obj=n_obj, p_real=P, rows=rows,
                             num_images=B)
    acc = pl.pallas_call(
        body,
        grid=(B // g,),
        in_specs=[
            pl.BlockSpec((g, 4, rows, _LANES), lambda i: (i, 0, 0, 0)),
            pl.BlockSpec((g, 2, rows, _LANES), lambda i: (i, 0, 0, 0)),
            pl.BlockSpec((g, n_obj, 5), lambda i: (i, 0, 0),
                         memory_space=pltpu.SMEM),
            pl.BlockSpec((9, rows, _LANES), lambda i: (0, 0, 0)),
        ],
        out_specs=pl.BlockSpec((4,), lambda i: (0,), memory_space=pltpu.SMEM),
        out_shape=jax.ShapeDtypeStruct((4,), jnp.float32),
        scratch_shapes=[
            pltpu.VMEM((B, rows, _LANES), jnp.int32),
            pltpu.SMEM((B,), jnp.int32),
            pltpu.SMEM((B,), jnp.int32),
        ],
    )(loc_p, conf_p, targets, planes)

    n = jnp.maximum(acc[3], 1.0)
    return acc[0] / n, (acc[1] + acc[2]) / n


# speedup vs baseline: 21.6263x; 1.0182x over previous
import functools

import jax
import jax.numpy as jnp
from jax.experimental import pallas as pl
from jax.experimental.pallas import tpu as pltpu

_OVERLAP_TH = 0.35
_VAR0, _VAR1 = 0.1, 0.2
_NP_RATIO = 3
_LANES = 128
_IMGS_PER_STEP = 2


def _body(loc_ref, conf_ref, tgt_ref, pln_ref, out_ref, rank_ref, kk_ref,
          pref_ref, iou_ref, *, n_obj, p_real, rows, num_images):
    step = pl.program_id(0)

    ones_row = jnp.ones((1, rows), jnp.float32)

    def msum(x):
        return jnp.sum(jax.lax.dot_general(
            ones_row, x, (((1,), (0,)), ((), ())),
            preferred_element_type=jnp.float32))

    px1 = pln_ref[0]
    py1 = pln_ref[1]
    px2 = pln_ref[2]
    py2 = pln_ref[3]
    parea = pln_ref[8]
    pcx = pln_ref[4]
    pcy = pln_ref[5]
    pw = pln_ref[6]
    ph = pln_ref[7]

    lane = jax.lax.broadcasted_iota(jnp.int32, (rows, _LANES), 1)
    row = jax.lax.broadcasted_iota(jnp.int32, (rows, _LANES), 0)
    lin = row * _LANES + lane

    @pl.when(step == 0)
    def _():
        out_ref[0] = 0.0
        out_ref[1] = 0.0
        out_ref[2] = 0.0
        out_ref[3] = 0.0

    best_ov_l = [jnp.full((rows, _LANES), -2.0, jnp.float32)
                 for _ in range(_IMGS_PER_STEP)]
    best_j_l = [jnp.zeros((rows, _LANES), jnp.int32)
                for _ in range(_IMGS_PER_STEP)]
    for j in range(n_obj):
        for im in range(_IMGS_PER_STEP):
            tx1 = tgt_ref[im, j, 0]
            ty1 = tgt_ref[im, j, 1]
            tx2 = tgt_ref[im, j, 2]
            ty2 = tgt_ref[im, j, 3]
            iw = jnp.maximum(jnp.minimum(tx2, px2) - jnp.maximum(tx1, px1), 0.0)
            ih = jnp.maximum(jnp.minimum(ty2, py2) - jnp.maximum(ty1, py1), 0.0)
            inter = iw * ih
            tarea = (tx2 - tx1) * (ty2 - ty1)
            iou = inter / (tarea + parea - inter)
            iou_ref[im, j] = iou
            upd = iou > best_ov_l[im]
            best_ov_l[im] = jnp.where(upd, iou, best_ov_l[im])
            best_j_l[im] = jnp.where(upd, j, best_j_l[im])

    bp_all = []
    for im in range(_IMGS_PER_STEP):
        bp_idx = []
        for j in range(n_obj):
            iou = iou_ref[im, j]
            mj = jnp.max(iou)
            bp_idx.append(jnp.min(jnp.where(iou == mj, lin, p_real)))
        bp_all.append(bp_idx)

    for im in range(_IMGS_PER_STEP):
        best_ov = best_ov_l[im]
        best_j = best_j_l[im]
        bp_idx = bp_all[im]

        for j in range(n_obj):
            f = lin == bp_idx[j]
            best_ov = jnp.where(f, 2.0, best_ov)
            best_j = jnp.where(f, j, best_j)

        m1 = jnp.zeros((rows, _LANES), jnp.float32)
        m2 = jnp.zeros((rows, _LANES), jnp.float32)
        m3 = jnp.ones((rows, _LANES), jnp.float32)
        m4 = jnp.ones((rows, _LANES), jnp.float32)
        lab = jnp.zeros((rows, _LANES), jnp.float32)
        for j in range(n_obj):
            s = best_j == j
            m1 = jnp.where(s, tgt_ref[im, j, 0], m1)
            m2 = jnp.where(s, tgt_ref[im, j, 1], m2)
            m3 = jnp.where(s, tgt_ref[im, j, 2], m3)
            m4 = jnp.where(s, tgt_ref[im, j, 3], m4)
            lab = jnp.where(s, tgt_ref[im, j, 4], lab)

        conf = jnp.where(best_ov < _OVERLAP_TH, 0.0, lab)
        pos = conf > 0.0
        npos = msum(pos.astype(jnp.float32))

        gcx = ((m1 + m3) * 0.5 - pcx) / (_VAR0 * pw)
        gcy = ((m2 + m4) * 0.5 - pcy) / (_VAR0 * ph)
        gw = jnp.log((m3 - m1) / pw) / _VAR1
        gh = jnp.log((m4 - m2) / ph) / _VAR1

        def sl1(d):
            ad = jnp.abs(d)
            return jnp.where(ad < 1.0, 0.5 * d * d, ad - 0.5)

        hub = (sl1(loc_ref[im, 0] - gcx) + sl1(loc_ref[im, 1] - gcy)
               + sl1(loc_ref[im, 2] - gw) + sl1(loc_ref[im, 3] - gh))
        loss_l = msum(jnp.where(pos, hub, 0.0))

        c0 = conf_ref[im, 0]
        c1 = conf_ref[im, 1]
        mx = jnp.maximum(c0, c1)
        lse = mx + jnp.log(1.0 + jnp.exp(jnp.minimum(c0, c1) - mx))
        ce_pos = msum(jnp.where(pos, lse - c1, 0.0))

        rank = jnp.where(pos, 0.0, lse - c0)
        img = step * _IMGS_PER_STEP + im
        rank_ref[img] = jax.lax.bitcast_convert_type(rank, jnp.int32)
        npos_i = npos.astype(jnp.int32)
        kk_ref[img] = jnp.minimum(_NP_RATIO * npos_i, p_real - 1)

        out_ref[0] += loss_l
        out_ref[1] += ce_pos
        out_ref[3] += npos

    @pl.when(step == num_images // _IMGS_PER_STEP - 1)
    def _():
        for im in range(num_images):
            pref_ref[im] = 0

        def bit_step(b, carry):
            bit = jnp.int32(1) << (30 - b)
            for im in range(num_images):
                pref = pref_ref[im]
                cand = pref | bit
                ge = (rank_ref[im] >= cand).astype(jnp.float32)
                cnt = msum(ge)
                pref_ref[im] = jnp.where(cnt >= kk_ref[im].astype(jnp.float32),
                                         cand, pref)
            return carry

        jax.lax.fori_loop(0, 31, bit_step, jnp.int32(0))

        neg_total = jnp.float32(0.0)
        for im in range(num_images):
            pref = pref_ref[im]
            rbv = rank_ref[im]
            gt = rbv > pref
            cnt_gt = msum(gt.astype(jnp.float32))
            rv = jax.lax.bitcast_convert_type(rbv, jnp.float32)
            sum_gt = msum(jnp.where(gt, rv, 0.0))
            t = jax.lax.bitcast_convert_type(pref, jnp.float32)
            kkf = kk_ref[im].astype(jnp.float32)
            neg_total += jnp.where(kk_ref[im] > 0,
                                   sum_gt + (kkf - cnt_gt) * t, 0.0)
        out_ref[2] = neg_total


def kernel(loc_data, conf_data, targets, priors):
    B, P, _ = loc_data.shape
    n_obj = targets.shape[1]
    rows = (P + _LANES - 1) // _LANES
    rows = ((rows + 7) // 8) * 8
    pp = rows * _LANES
    pad = pp - P

    pad_cf = jnp.tile(jnp.asarray([[2.5, 2.5, 1.0, 1.0]], priors.dtype), (pad, 1))
    pr = jnp.concatenate([priors, pad_cf], axis=0)
    pf = jnp.concatenate([pr[:, :2] - pr[:, 2:] / 2.0,
                          pr[:, :2] + pr[:, 2:] / 2.0], axis=1)
    parea = (pf[:, 2] - pf[:, 0]) * (pf[:, 3] - pf[:, 1])
    planes = jnp.concatenate([pf.T, pr.T, parea[None, :]], axis=0)
    planes = planes.reshape(9, rows, _LANES)

    loc_p = jnp.pad(jnp.moveaxis(loc_data, -1, 1), ((0, 0), (0, 0), (0, pad)))
    loc_p = loc_p.reshape(B, 4, rows, _LANES)
    conf_pad = jnp.broadcast_to(
        jnp.asarray([40.0, -40.0], conf_data.dtype)[None, :, None], (B, 2, pad))
    conf_p = jnp.concatenate([jnp.moveaxis(conf_data, -1, 1), conf_pad], axis=2)
    conf_p = conf_p.reshape(B, 2, rows, _LANES)

    g = _IMGS_PER_STEP
    body = functools.partial(_body, n_obj=n_obj, p_real=P, rows=rows,
                             num_images=B)
    acc = pl.pallas_call(
        body,
        grid=(B // g,),
        in_specs=[
            pl.BlockSpec((g, 4, rows, _LANES), lambda i: (i, 0, 0, 0)),
            pl.BlockSpec((g, 2, rows, _LANES), lambda i: (i, 0, 0, 0)),
            pl.BlockSpec((g, n_obj, 5), lambda i: (i, 0, 0),
                         memory_space=pltpu.SMEM),
            pl.BlockSpec((9, rows, _LANES), lambda i: (0, 0, 0)),
        ],
        out_specs=pl.BlockSpec((4,), lambda i: (0,), memory_space=pltpu.SMEM),
        out_shape=jax.ShapeDtypeStruct((4,), jnp.float32),
        scratch_shapes=[
            pltpu.VMEM((B, rows, _LANES), jnp.int32),
            pltpu.SMEM((B,), jnp.int32),
            pltpu.SMEM((B,), jnp.int32),
            pltpu.VMEM((g, n_obj, rows, _LANES), jnp.float32),
        ],
    )(loc_p, conf_p, targets, planes)

    n = jnp.maximum(acc[3], 1.0)
    return acc[0] / n, (acc[1] + acc[2]) / n


# speedup vs baseline: 29.9708x; 1.3858x over previous
import functools

import jax
import jax.numpy as jnp
from jax.experimental import pallas as pl
from jax.experimental.pallas import tpu as pltpu

_OVERLAP_TH = 0.35
_VAR0, _VAR1 = 0.1, 0.2
_NP_RATIO = 3
_LANES = 128
_IMGS_PER_STEP = 2


def _body(loc_ref, conf_ref, tgt_ref, pln_ref, out_ref, rank_ref, kk_ref,
          pref_ref, *, n_obj, p_real, rows, num_images):
    step = pl.program_id(0)

    ones_row = jnp.ones((1, rows), jnp.float32)

    def msum(x):
        return jnp.sum(jax.lax.dot_general(
            ones_row, x, (((1,), (0,)), ((), ())),
            preferred_element_type=jnp.float32))

    px1 = pln_ref[0]
    py1 = pln_ref[1]
    px2 = pln_ref[2]
    py2 = pln_ref[3]
    parea = pln_ref[8]
    pcx = pln_ref[4]
    pcy = pln_ref[5]
    pw = pln_ref[6]
    ph = pln_ref[7]

    lane = jax.lax.broadcasted_iota(jnp.int32, (rows, _LANES), 1)
    row = jax.lax.broadcasted_iota(jnp.int32, (rows, _LANES), 0)
    lin = row * _LANES + lane

    @pl.when(step == 0)
    def _():
        out_ref[0] = 0.0
        out_ref[1] = 0.0
        out_ref[2] = 0.0
        out_ref[3] = 0.0

    best_ov_l = [jnp.full((rows, _LANES), -2.0, jnp.float32)
                 for _ in range(_IMGS_PER_STEP)]
    best_j_l = [jnp.zeros((rows, _LANES), jnp.int32)
                for _ in range(_IMGS_PER_STEP)]
    smax_l = [[] for _ in range(_IMGS_PER_STEP)]
    clin_l = [[] for _ in range(_IMGS_PER_STEP)]
    for j in range(n_obj):
        for im in range(_IMGS_PER_STEP):
            tx1 = tgt_ref[im, j, 0]
            ty1 = tgt_ref[im, j, 1]
            tx2 = tgt_ref[im, j, 2]
            ty2 = tgt_ref[im, j, 3]
            iw = jnp.maximum(jnp.minimum(tx2, px2) - jnp.maximum(tx1, px1), 0.0)
            ih = jnp.maximum(jnp.minimum(ty2, py2) - jnp.maximum(ty1, py1), 0.0)
            inter = iw * ih
            tarea = (tx2 - tx1) * (ty2 - ty1)
            iou = inter / (tarea + parea - inter)
            upd = iou > best_ov_l[im]
            best_ov_l[im] = jnp.where(upd, iou, best_ov_l[im])
            best_j_l[im] = jnp.where(upd, j, best_j_l[im])
            smax = jnp.max(iou, axis=0, keepdims=True)
            smax_l[im].append(smax)
            clin_l[im].append(jnp.min(jnp.where(iou == smax, lin, p_real),
                                      axis=0, keepdims=True))

    bp_rows = []
    for im in range(_IMGS_PER_STEP):
        s_slab = jnp.concatenate(smax_l[im], axis=0)
        c_slab = jnp.concatenate(clin_l[im], axis=0)
        mj = jnp.max(s_slab, axis=1, keepdims=True)
        bp = jnp.min(jnp.where(s_slab == mj, c_slab, p_real),
                     axis=1, keepdims=True)
        bp_rows.append(jnp.broadcast_to(bp, (n_obj, _LANES)))

    for im in range(_IMGS_PER_STEP):
        best_ov = best_ov_l[im]
        best_j = best_j_l[im]

        for j in range(n_obj):
            f = lin == bp_rows[im][j:j + 1, :]
            best_ov = jnp.where(f, 2.0, best_ov)
            best_j = jnp.where(f, j, best_j)

        m1 = jnp.zeros((rows, _LANES), jnp.float32)
        m2 = jnp.zeros((rows, _LANES), jnp.float32)
        m3 = jnp.ones((rows, _LANES), jnp.float32)
        m4 = jnp.ones((rows, _LANES), jnp.float32)
        lab = jnp.zeros((rows, _LANES), jnp.float32)
        for j in range(n_obj):
            s = best_j == j
            m1 = jnp.where(s, tgt_ref[im, j, 0], m1)
            m2 = jnp.where(s, tgt_ref[im, j, 1], m2)
            m3 = jnp.where(s, tgt_ref[im, j, 2], m3)
            m4 = jnp.where(s, tgt_ref[im, j, 3], m4)
            lab = jnp.where(s, tgt_ref[im, j, 4], lab)

        conf = jnp.where(best_ov < _OVERLAP_TH, 0.0, lab)
        pos = conf > 0.0
        npos = msum(pos.astype(jnp.float32))

        gcx = ((m1 + m3) * 0.5 - pcx) / (_VAR0 * pw)
        gcy = ((m2 + m4) * 0.5 - pcy) / (_VAR0 * ph)
        gw = jnp.log((m3 - m1) / pw) / _VAR1
        gh = jnp.log((m4 - m2) / ph) / _VAR1

        def sl1(d):
            ad = jnp.abs(d)
            return jnp.where(ad < 1.0, 0.5 * d * d, ad - 0.5)

        hub = (sl1(loc_ref[im, 0] - gcx) + sl1(loc_ref[im, 1] - gcy)
               + sl1(loc_ref[im, 2] - gw) + sl1(loc_ref[im, 3] - gh))
        loss_l = msum(jnp.where(pos, hub, 0.0))

        c0 = conf_ref[im, 0]
        c1 = conf_ref[im, 1]
        mx = jnp.maximum(c0, c1)
        lse = mx + jnp.log(1.0 + jnp.exp(jnp.minimum(c0, c1) - mx))
        ce_pos = msum(jnp.where(pos, lse - c1, 0.0))

        rank = jnp.where(pos, 0.0, lse - c0)
        img = step * _IMGS_PER_STEP + im
        rank_ref[img] = jax.lax.bitcast_convert_type(rank, jnp.int32)
        npos_i = npos.astype(jnp.int32)
        kk_ref[img] = jnp.minimum(_NP_RATIO * npos_i, p_real - 1)

        out_ref[0] += loss_l
        out_ref[1] += ce_pos
        out_ref[3] += npos

    @pl.when(step == num_images // _IMGS_PER_STEP - 1)
    def _():
        for im in range(num_images):
            pref_ref[im] = 0

        def bit_step(b, carry):
            bit = jnp.int32(1) << (30 - b)
            for im in range(num_images):
                pref = pref_ref[im]
                cand = pref | bit
                ge = (rank_ref[im] >= cand).astype(jnp.float32)
                cnt = msum(ge)
                pref_ref[im] = jnp.where(cnt >= kk_ref[im].astype(jnp.float32),
                                         cand, pref)
            return carry

        jax.lax.fori_loop(0, 31, bit_step, jnp.int32(0))

        neg_total = jnp.float32(0.0)
        for im in range(num_images):
            pref = pref_ref[im]
            rbv = rank_ref[im]
            gt = rbv > pref
            cnt_gt = msum(gt.astype(jnp.float32))
            rv = jax.lax.bitcast_convert_type(rbv, jnp.float32)
            sum_gt = msum(jnp.where(gt, rv, 0.0))
            t = jax.lax.bitcast_convert_type(pref, jnp.float32)
            kkf = kk_ref[im].astype(jnp.float32)
            neg_total += jnp.where(kk_ref[im] > 0,
                                   sum_gt + (kkf - cnt_gt) * t, 0.0)
        out_ref[2] = neg_total


def kernel(loc_data, conf_data, targets, priors):
    B, P, _ = loc_data.shape
    n_obj = targets.shape[1]
    rows = (P + _LANES - 1) // _LANES
    rows = ((rows + 7) // 8) * 8
    pp = rows * _LANES
    pad = pp - P

    pad_cf = jnp.tile(jnp.asarray([[2.5, 2.5, 1.0, 1.0]], priors.dtype), (pad, 1))
    pr = jnp.concatenate([priors, pad_cf], axis=0)
    pf = jnp.concatenate([pr[:, :2] - pr[:, 2:] / 2.0,
                          pr[:, :2] + pr[:, 2:] / 2.0], axis=1)
    parea = (pf[:, 2] - pf[:, 0]) * (pf[:, 3] - pf[:, 1])
    planes = jnp.concatenate([pf.T, pr.T, parea[None, :]], axis=0)
    planes = planes.reshape(9, rows, _LANES)

    loc_p = jnp.pad(jnp.moveaxis(loc_data, -1, 1), ((0, 0), (0, 0), (0, pad)))
    loc_p = loc_p.reshape(B, 4, rows, _LANES)
    conf_pad = jnp.broadcast_to(
        jnp.asarray([40.0, -40.0], conf_data.dtype)[None, :, None], (B, 2, pad))
    conf_p = jnp.concatenate([jnp.moveaxis(conf_data, -1, 1), conf_pad], axis=2)
    conf_p = conf_p.reshape(B, 2, rows, _LANES)

    g = _IMGS_PER_STEP
    body = functools.partial(_body, n_obj=n_obj, p_real=P, rows=rows,
                             num_images=B)
    acc = pl.pallas_call(
        body,
        grid=(B // g,),
        in_specs=[
            pl.BlockSpec((g, 4, rows, _LANES), lambda i: (i, 0, 0, 0)),
            pl.BlockSpec((g, 2, rows, _LANES), lambda i: (i, 0, 0, 0)),
            pl.BlockSpec((g, n_obj, 5), lambda i: (i, 0, 0),
                         memory_space=pltpu.SMEM),
            pl.BlockSpec((9, rows, _LANES), lambda i: (0, 0, 0)),
        ],
        out_specs=pl.BlockSpec((4,), lambda i: (0,), memory_space=pltpu.SMEM),
        out_shape=jax.ShapeDtypeStruct((4,), jnp.float32),
        scratch_shapes=[
            pltpu.VMEM((B, rows, _LANES), jnp.int32),
            pltpu.SMEM((B,), jnp.int32),
            pltpu.SMEM((B,), jnp.int32),
        ],
    )(loc_p, conf_p, targets, planes)

    n = jnp.maximum(acc[3], 1.0)
    return acc[0] / n, (acc[1] + acc[2]) / n
